# baseline (speedup 1.0000x reference)
import jax
import jax.numpy as jnp
from jax import lax
from jax.experimental import pallas as pl
from jax.experimental.pallas import tpu as pltpu
from jax.experimental.pallas import tpu_sc as plsc

N = 10000
E = 320000
NF = 128
EF = 16
H = 128

KB = 64
NS = 16
NC = 2
EPW = E // NS
RPW = 320
JC = 8
NNODE = 10240
NBLKN = NNODE // KB
BN = 1024

_PREC = lax.Precision.HIGHEST


def _dot(a, b):
    return jnp.dot(a, b, precision=_PREC, preferred_element_type=jnp.float32)


def _make_edge_op():
    scratch = [
        pltpu.VMEM((JC, KB), jnp.int32),
        pltpu.VMEM((JC, KB), jnp.int32),
        pltpu.VMEM((JC, KB), jnp.int32),
        pltpu.VMEM((16,), jnp.int32),
        pltpu.VMEM((16,), jnp.float32),
        pltpu.VMEM((16,), jnp.float32),
        pltpu.VMEM((16,), jnp.float32),
        pltpu.VMEM((16,), jnp.float32),
        pltpu.VMEM((KB, H), jnp.float32),
        pltpu.VMEM((KB, H), jnp.float32),
        pltpu.VMEM((KB, H), jnp.float32),
        pltpu.VMEM((KB, H), jnp.float32),
        pltpu.VMEM((KB, H), jnp.float32),
        pltpu.VMEM_SHARED((NNODE, H), jnp.float32),
        pltpu.SemaphoreType.DMA,
        pltpu.SemaphoreType.DMA,
    ]

    def body(a_hbm, b_hbm, c_hbm, src_hbm, dst_hbm, cid_hbm, m2_hbm,
             ga_hbm, gc_hbm, al_hbm, be_hbm, out_hbm,
             sidx, didx, cidx, m2_v, ga_v, gc_v, al_v, be_v,
             a_v, b_v, c_v, v_v, w_v, acc, sem_g, sem_s):
        cid = lax.axis_index("c")
        sid = lax.axis_index("s")

        pltpu.sync_copy(m2_hbm, m2_v)
        pltpu.sync_copy(ga_hbm, ga_v)
        pltpu.sync_copy(gc_hbm, gc_v)
        pltpu.sync_copy(al_hbm, al_v)
        pltpu.sync_copy(be_hbm, be_v)

        def zrow(r, carry):
            for q in range(H // 16):
                v_v[r, pl.ds(q * 16, 16)] = jnp.zeros((16,), jnp.float32)
            return carry
        lax.fori_loop(0, KB, zrow, 0)

        zd = [pltpu.async_copy(v_v, acc.at[pl.ds((sid + i * NS) * KB, KB), :],
                               sem_g) for i in range(NBLKN // NS)]
        for d in zd:
            d.wait()

        cid0 = lax.convert_element_type(cid == 0, jnp.int32)
        active = jnp.maximum(m2_v[...], jnp.broadcast_to(cid0, (16,)))
        inactive_n = (1 - active) * N

        plsc.subcore_barrier()
        ga = ga_v[...]
        gc = gc_v[...]
        alpha = al_v[...]
        beta = be_v[...]

        def echunk(jo, carry):
            rows = pl.ds(jo * JC, JC)

            @pl.when(cid == 0)
            def _():
                pltpu.async_copy(src_hbm.at[sid, rows, :], sidx, sem_g)
                pltpu.async_copy(dst_hbm.at[sid, rows, :], didx, sem_g)

            @pl.when(cid != 0)
            def _():
                pltpu.async_copy(dst_hbm.at[sid, rows, :], sidx, sem_g)
                pltpu.async_copy(src_hbm.at[sid, rows, :], didx, sem_g)

            pltpu.async_copy(cid_hbm.at[sid, rows, :], cidx, sem_g).wait()
            pltpu.make_async_copy(cid_hbm.at[sid, rows, :], cidx, sem_g).wait()
            pltpu.make_async_copy(cid_hbm.at[sid, rows, :], cidx, sem_g).wait()

            def mrow(r, c2):
                for q in range(KB // 16):
                    sl = pl.ds(q * 16, 16)
                    didx[r, sl] = active * didx[r, sl] + inactive_n
                return c2
            lax.fori_loop(0, JC, mrow, 0)

            sdesc = [None, None]
            for k in range(JC):
                d0 = pltpu.async_copy(a_hbm.at[didx.at[k]], a_v, sem_g)
                d1 = pltpu.async_copy(b_hbm.at[sidx.at[k]], b_v, sem_g)
                d2 = pltpu.async_copy(c_hbm.at[cidx.at[k]], c_v, sem_g)
                d0.wait()
                d1.wait()
                d2.wait()
                vbuf = v_v if k % 2 == 0 else w_v
                if sdesc[k % 2] is not None:
                    sdesc[k % 2].wait()

                def crow(r, c3, vbuf=vbuf):
                    for q in range(H // 16):
                        sl = pl.ds(q * 16, 16)
                        xv = ga * a_v[r, sl] + b_v[r, sl] + gc * c_v[r, sl]
                        vbuf[r, sl] = alpha * jnp.maximum(xv, 0.0) + beta * xv
                    return c3
                lax.fori_loop(0, KB, crow, 0, unroll=4)

                sdesc[k % 2] = pltpu.async_copy(vbuf, acc.at[didx.at[k]],
                                                sem_s, add=True)
            sdesc[0].wait()
            sdesc[1].wait()
            return carry
        lax.fori_loop(0, RPW // JC, echunk, 0)

        plsc.subcore_barrier()

        dd = []
        for i in range(NBLKN // NS):
            r0 = (sid + i * NS) * KB
            dd.append(pltpu.async_copy(
                acc.at[pl.ds(r0, KB), :],
                out_hbm.at[pl.ds(cid * NNODE + r0, KB), :], sem_g))
        for d in dd:
            d.wait()

    return pl.kernel(body,
                     out_type=jax.ShapeDtypeStruct((NC * NNODE, H),
                                                   jnp.float32),
                     mesh=plsc.VectorSubcoreMesh(core_axis_name="c",
                                                 subcore_axis_name="s"),
                     scratch_types=tuple(scratch))


def _dir_body(s0r, d0r, src_r, dst_r, out):
    i = pl.program_id(0)

    @pl.when(i == 0)
    def _():
        out[...] = jnp.ones((8, 128), jnp.int32)

    hit = jnp.any((src_r[...] == d0r[0, 0]) & (dst_r[...] == s0r[0, 0]))

    @pl.when(hit)
    def _():
        out[...] = jnp.zeros((8, 128), jnp.int32)


def _directed_flag(edge_index):
    DB = 20
    src_r = edge_index[0].reshape(DB, E // DB // 128, 128)
    dst_r = edge_index[1].reshape(DB, E // DB // 128, 128)
    s0 = edge_index[0, 0].reshape(1, 1)
    d0 = edge_index[1, 0].reshape(1, 1)
    blk = pl.BlockSpec((1, E // DB // 128, 128), lambda i: (i, 0, 0))
    smem = pl.BlockSpec(memory_space=pltpu.SMEM)
    out = pl.pallas_call(
        _dir_body,
        grid=(DB,),
        in_specs=[smem, smem, blk, blk],
        out_specs=pl.BlockSpec((8, 128), lambda i: (0, 0)),
        out_shape=jax.ShapeDtypeStruct((8, 128), jnp.int32),
    )(s0, d0, src_r, dst_r)
    return out[0, :16]


def _c_body(ea, w, b, c):
    c[...] = _dot(ea[...], w[...]) + b[0]


def _prep_c(ea, wb_pairs):
    EB = 4000
    nb = E // EB
    wstack = jnp.concatenate([w for w, _ in wb_pairs], axis=0)
    bstack = jnp.stack([b.reshape(1, H) for _, b in wb_pairs], axis=0)
    return pl.pallas_call(
        _c_body,
        grid=(3 * nb,),
        in_specs=[pl.BlockSpec((EB, EF), lambda i: (i % nb, 0)),
                  pl.BlockSpec((EF, H), lambda i: (i // nb, 0)),
                  pl.BlockSpec((1, 1, H), lambda i: (i // nb, 0, 0))],
        out_specs=pl.BlockSpec((EB, H), lambda i: (i, 0)),
        out_shape=jax.ShapeDtypeStruct((3 * E, H), jnp.float32),
    )(ea, wstack, bstack)


def _hspec():
    return pl.BlockSpec((BN, H), lambda i: (i, 0))


def _wspec():
    return pl.BlockSpec((H, H), lambda i: (0, 0))


def _bspec():
    return pl.BlockSpec((1, H), lambda i: (0, 0))


def _ab_body(h, wd, ws, a, b):
    hh = h[...]
    a[...] = _dot(hh, wd[...])
    b[...] = _dot(hh, ws[...])


def _ab(h, w1):
    return pl.pallas_call(
        _ab_body,
        grid=(NNODE // BN,),
        in_specs=[_hspec(), _wspec(), _wspec()],
        out_specs=[_hspec(), _hspec()],
        out_shape=[jax.ShapeDtypeStruct((NNODE, H), jnp.float32)] * 2,
    )(h, w1[:H], w1[H:2 * H])


def _deg_dis(da, db):
    deg = (da[...] + db[...])[:, :1]
    dis = jnp.where(deg > 0, lax.rsqrt(deg), 0.0)
    return deg, dis


def _post_body(sa, sb, da, db, w2, b2, tw0, g, hs):
    deg, dis = _deg_dis(da, db)
    h1 = jnp.maximum(_dot(sa[...] + sb[...], w2[...]) + deg * b2[...], 0.0)
    g[...] = _dot(h1, tw0[...])
    hs[...] = dis * h1


def _post(sa, sb, da, db, w2, b2, tw0):
    return pl.pallas_call(
        _post_body,
        grid=(NNODE // BN,),
        in_specs=[_hspec()] * 4 + [_wspec(), _bspec(), _wspec()],
        out_specs=[_hspec(), _hspec()],
        out_shape=[jax.ShapeDtypeStruct((NNODE, H), jnp.float32)] * 2,
    )(sa, sb, da, db, w2, b2.reshape(1, H), tw0)


def _mid_body(pa, pb, da, db, tw1, term, hs2):
    _, dis = _deg_dis(da, db)
    hp = dis * (pa[...] + pb[...])
    term[...] = _dot(hp, tw1[...])
    hs2[...] = dis * hp


def _mid(pa, pb, da, db, tw1):
    return pl.pallas_call(
        _mid_body,
        grid=(NNODE // BN,),
        in_specs=[_hspec()] * 4 + [_wspec()],
        out_specs=[_hspec(), _hspec()],
        out_shape=[jax.ShapeDtypeStruct((NNODE, H), jnp.float32)] * 2,
    )(pa, pb, da, db, tw1)


def _tagfin_body(pa, pb, da, db, g, term, tw2, tb, wd, ws, a, b):
    _, dis = _deg_dis(da, db)
    hp = dis * (pa[...] + pb[...])
    ht = jnp.maximum(g[...] + term[...] + _dot(hp, tw2[...]) + tb[...], 0.0)
    a[...] = _dot(ht, wd[...])
    b[...] = _dot(ht, ws[...])


def _tagfin(pa, pb, da, db, g, term, tw2, tb, w1_next):
    return pl.pallas_call(
        _tagfin_body,
        grid=(NNODE // BN,),
        in_specs=[_hspec()] * 6 + [_wspec(), _bspec(), _wspec(), _wspec()],
        out_specs=[_hspec(), _hspec()],
        out_shape=[jax.ShapeDtypeStruct((NNODE, H), jnp.float32)] * 2,
    )(pa, pb, da, db, g, term, tw2, tb.reshape(1, H), w1_next[:H],
      w1_next[H:2 * H])


def _final_body(sa, sb, da, db, w2, b2, out):
    deg, _ = _deg_dis(da, db)
    out[...] = _dot(sa[...] + sb[...], w2[...]) + deg * b2[...]


def _final(sa, sb, da, db, w2, b2):
    return pl.pallas_call(
        _final_body,
        grid=(NNODE // BN,),
        in_specs=[_hspec()] * 4 + [_wspec(), _bspec()],
        out_specs=_hspec(),
        out_shape=jax.ShapeDtypeStruct((NNODE, H), jnp.float32),
    )(sa, sb, da, db, w2, b2.reshape(1, H))


def kernel(x, edge_index, edge_attr,
           l0_w1, l0_b1, l0_w2, l0_b2,
           t1_w0, t1_w1, t1_w2, t1_b,
           l2_w1, l2_b1, l2_w2, l2_b2,
           t3_w0, t3_w1, t3_w2, t3_b,
           l4_w1, l4_b1, l4_w2, l4_b2):
    h0 = jnp.pad(x[:, 4:4 + NF], ((0, NNODE - N), (0, 0)))

    def pad_edges(v, fill):
        return jnp.pad(v.reshape(NS, EPW), ((0, 0), (0, RPW * KB - EPW)),
                       constant_values=fill).reshape(NS, RPW, KB)

    srcm = pad_edges(edge_index[0], N)
    dstm = pad_edges(edge_index[1], N)

    m2 = _directed_flag(edge_index)
    ones_n = jnp.ones((NNODE, H), jnp.float32)
    zn = jnp.zeros((NNODE, H), jnp.float32)

    cidx_base = pad_edges(jnp.arange(E, dtype=jnp.int32), 0)
    cidx_hot = jnp.broadcast_to(jnp.arange(KB, dtype=jnp.int32),
                                (NS, RPW, KB))

    def coef(v):
        return jnp.full((16,), v, jnp.float32)

    MSG = (coef(1.0), coef(1.0), coef(1.0), coef(0.0))
    HOP = (coef(0.0), coef(0.0), coef(0.0), coef(1.0))

    cbig = _prep_c(edge_attr, [(l0_w1[2 * H:], l0_b1), (l2_w1[2 * H:], l2_b1),
                               (l4_w1[2 * H:], l4_b1)])
    a0, b0 = _ab(h0, l0_w1)

    edge_op = _make_edge_op()

    def split(p):
        return p[:NNODE], p[NNODE:]

    def st0(p, g, term, da, db):
        return (ones_n, ones_n, cidx_hot) + HOP + (g, term, da, db)

    def st1(p, g, term, da, db):
        da, db = split(p)
        return (a0, b0, cidx_base) + MSG + (g, term, da, db)

    def _post_stage(p, g, term, da, db, w2, b2, tw0):
        g, hs = _post(p[:NNODE], p[NNODE:], da, db, w2, b2, tw0)
        return (hs, hs, cidx_hot) + HOP + (g, term, da, db)

    def _mid_stage(p, g, term, da, db, tw1):
        term, hs2 = _mid(p[:NNODE], p[NNODE:], da, db, tw1)
        return (hs2, hs2, cidx_hot) + HOP + (g, term, da, db)

    def _fin_stage(p, g, term, da, db, tw2, tb, w1n, lyr):
        a, b = _tagfin(p[:NNODE], p[NNODE:], da, db, g, term, tw2, tb, w1n)
        return (a, b, cidx_base + lyr * E) + MSG + (g, term, da, db)

    branches = [
        st0,
        st1,
        lambda p, g, t, da, db: _post_stage(p, g, t, da, db, l0_w2, l0_b2,
                                            t1_w0),
        lambda p, g, t, da, db: _mid_stage(p, g, t, da, db, t1_w1),
        lambda p, g, t, da, db: _fin_stage(p, g, t, da, db, t1_w2, t1_b,
                                           l2_w1, 1),
        lambda p, g, t, da, db: _post_stage(p, g, t, da, db, l2_w2, l2_b2,
                                            t3_w0),
        lambda p, g, t, da, db: _mid_stage(p, g, t, da, db, t3_w1),
        lambda p, g, t, da, db: _fin_stage(p, g, t, da, db, t3_w2, t3_b,
                                           l4_w1, 2),
    ]

    def body(carry, i):
        p, g, term, da, db = carry
        a, b, cidx, ga, gc, al, be, g, term, da, db = lax.switch(
            i, branches, p, g, term, da, db)
        p = edge_op(a, b, cbig, srcm, dstm, cidx, m2, ga, gc, al, be)
        return (p, g, term, da, db), None

    carry0 = (jnp.zeros((NC * NNODE, H), jnp.float32), zn, zn, zn, zn)
    (p, _, _, da, db), _ = lax.scan(body, carry0,
                                    jnp.arange(8, dtype=jnp.int32))
    return _final(p[:NNODE], p[NNODE:], da, db, l4_w2, l4_b2)[:N]


# speedup vs baseline: 1.7227x; 1.7227x over previous
import jax
import jax.numpy as jnp
from jax import lax
from jax.experimental import pallas as pl
from jax.experimental.pallas import tpu as pltpu
from jax.experimental.pallas import tpu_sc as plsc

N = 10000
E = 320000
NF = 128
EF = 16
H = 128

KB = 64
NS = 16
NC = 2
EPW = E // NS
RPW = 320
JC = 8
NNODE = 10240
NBLKN = NNODE // KB
BN = 1024

_PREC = lax.Precision.HIGHEST


def _dot(a, b):
    return jnp.dot(a, b, precision=_PREC, preferred_element_type=jnp.float32)


def _make_edge_op():
    scratch = [
        pltpu.VMEM((JC, KB), jnp.int32),
        pltpu.VMEM((JC, KB), jnp.int32),
        pltpu.VMEM((JC, KB), jnp.int32),
        pltpu.VMEM((16,), jnp.int32),
        pltpu.VMEM((16,), jnp.float32),
        pltpu.VMEM((16,), jnp.float32),
        pltpu.VMEM((16,), jnp.float32),
        pltpu.VMEM((16,), jnp.float32),
        pltpu.VMEM((KB, H), jnp.float32),
        pltpu.VMEM((KB, H), jnp.float32),
        pltpu.VMEM((KB, H), jnp.float32),
        pltpu.VMEM((KB, H), jnp.float32),
        pltpu.VMEM((KB, H), jnp.float32),
        pltpu.VMEM_SHARED((NNODE, H), jnp.float32),
        pltpu.SemaphoreType.DMA,
        pltpu.SemaphoreType.DMA,
    ]

    def body(a_hbm, b_hbm, c_hbm, src_hbm, dst_hbm, cid_hbm, m2_hbm,
             ga_hbm, gc_hbm, al_hbm, be_hbm, out_hbm,
             sidx, didx, cidx, m2_v, ga_v, gc_v, al_v, be_v,
             a_v, b_v, c_v, v_v, w_v, acc, sem_g, sem_s):
        cid = lax.axis_index("c")
        sid = lax.axis_index("s")

        pltpu.sync_copy(m2_hbm, m2_v)
        pltpu.sync_copy(ga_hbm, ga_v)
        pltpu.sync_copy(gc_hbm, gc_v)
        pltpu.sync_copy(al_hbm, al_v)
        pltpu.sync_copy(be_hbm, be_v)

        def zrow(r, carry):
            for q in range(H // 16):
                v_v[r, pl.ds(q * 16, 16)] = jnp.zeros((16,), jnp.float32)
            return carry
        lax.fori_loop(0, KB, zrow, 0)

        zd = [pltpu.async_copy(v_v, acc.at[pl.ds((sid + i * NS) * KB, KB), :],
                               sem_g) for i in range(NBLKN // NS)]
        for d in zd:
            d.wait()

        cid0 = lax.convert_element_type(cid == 0, jnp.int32)
        active = jnp.maximum(m2_v[...], jnp.broadcast_to(cid0, (16,)))
        inactive_n = (1 - active) * N

        plsc.subcore_barrier()
        ga = ga_v[...]
        gc = gc_v[...]
        alpha = al_v[...]
        beta = be_v[...]

        def echunk(jo, carry):
            rows = pl.ds(jo * JC, JC)

            @pl.when(cid == 0)
            def _():
                pltpu.async_copy(src_hbm.at[sid, rows, :], sidx, sem_g)
                pltpu.async_copy(dst_hbm.at[sid, rows, :], didx, sem_g)

            @pl.when(cid != 0)
            def _():
                pltpu.async_copy(dst_hbm.at[sid, rows, :], sidx, sem_g)
                pltpu.async_copy(src_hbm.at[sid, rows, :], didx, sem_g)

            pltpu.async_copy(cid_hbm.at[sid, rows, :], cidx, sem_g).wait()
            pltpu.make_async_copy(cid_hbm.at[sid, rows, :], cidx, sem_g).wait()
            pltpu.make_async_copy(cid_hbm.at[sid, rows, :], cidx, sem_g).wait()

            def mrow(r, c2):
                for q in range(KB // 16):
                    sl = pl.ds(q * 16, 16)
                    didx[r, sl] = active * didx[r, sl] + inactive_n
                return c2
            lax.fori_loop(0, JC, mrow, 0)

            sdesc = [None, None]
            for k in range(JC):
                d0 = pltpu.async_copy(a_hbm.at[didx.at[k]], a_v, sem_g)
                d1 = pltpu.async_copy(b_hbm.at[sidx.at[k]], b_v, sem_g)
                d2 = pltpu.async_copy(c_hbm.at[cidx.at[k]], c_v, sem_g)
                d0.wait()
                d1.wait()
                d2.wait()
                vbuf = v_v if k % 2 == 0 else w_v
                if sdesc[k % 2] is not None:
                    sdesc[k % 2].wait()

                def crow(r, c3, vbuf=vbuf):
                    for q in range(H // 16):
                        sl = pl.ds(q * 16, 16)
                        xv = ga * a_v[r, sl] + b_v[r, sl] + gc * c_v[r, sl]
                        vbuf[r, sl] = alpha * jnp.maximum(xv, 0.0) + beta * xv
                    return c3
                lax.fori_loop(0, KB, crow, 0)

                sdesc[k % 2] = pltpu.async_copy(vbuf, acc.at[didx.at[k]],
                                                sem_s, add=True)
            sdesc[0].wait()
            sdesc[1].wait()
            return carry
        lax.fori_loop(0, RPW // JC, echunk, 0)

        plsc.subcore_barrier()

        dd = []
        for i in range(NBLKN // NS):
            r0 = (sid + i * NS) * KB
            dd.append(pltpu.async_copy(
                acc.at[pl.ds(r0, KB), :],
                out_hbm.at[pl.ds(cid * NNODE + r0, KB), :], sem_g))
        for d in dd:
            d.wait()

    return pl.kernel(body,
                     out_type=jax.ShapeDtypeStruct((NC * NNODE, H),
                                                   jnp.float32),
                     mesh=plsc.VectorSubcoreMesh(core_axis_name="c",
                                                 subcore_axis_name="s"),
                     scratch_types=tuple(scratch))


def _dir_body(s0r, d0r, src_r, dst_r, out):
    i = pl.program_id(0)

    @pl.when(i == 0)
    def _():
        out[...] = jnp.ones((8, 128), jnp.int32)

    hit = jnp.any((src_r[...] == d0r[0, 0]) & (dst_r[...] == s0r[0, 0]))

    @pl.when(hit)
    def _():
        out[...] = jnp.zeros((8, 128), jnp.int32)


def _directed_flag(edge_index):
    DB = 20
    src_r = edge_index[0].reshape(DB, E // DB // 128, 128)
    dst_r = edge_index[1].reshape(DB, E // DB // 128, 128)
    s0 = edge_index[0, 0].reshape(1, 1)
    d0 = edge_index[1, 0].reshape(1, 1)
    blk = pl.BlockSpec((1, E // DB // 128, 128), lambda i: (i, 0, 0))
    smem = pl.BlockSpec(memory_space=pltpu.SMEM)
    out = pl.pallas_call(
        _dir_body,
        grid=(DB,),
        in_specs=[smem, smem, blk, blk],
        out_specs=pl.BlockSpec((8, 128), lambda i: (0, 0)),
        out_shape=jax.ShapeDtypeStruct((8, 128), jnp.int32),
    )(s0, d0, src_r, dst_r)
    return out[0, :16]


def _c_body(ea, w, b, c):
    c[...] = _dot(ea[...], w[...]) + b[0]


def _prep_c(ea, wb_pairs):
    EB = 4000
    nb = E // EB
    wstack = jnp.concatenate([w for w, _ in wb_pairs], axis=0)
    bstack = jnp.stack([b.reshape(1, H) for _, b in wb_pairs], axis=0)
    return pl.pallas_call(
        _c_body,
        grid=(3 * nb,),
        in_specs=[pl.BlockSpec((EB, EF), lambda i: (i % nb, 0)),
                  pl.BlockSpec((EF, H), lambda i: (i // nb, 0)),
                  pl.BlockSpec((1, 1, H), lambda i: (i // nb, 0, 0))],
        out_specs=pl.BlockSpec((EB, H), lambda i: (i, 0)),
        out_shape=jax.ShapeDtypeStruct((3 * E, H), jnp.float32),
    )(ea, wstack, bstack)


def _hspec():
    return pl.BlockSpec((BN, H), lambda i: (i, 0))


def _wspec():
    return pl.BlockSpec((H, H), lambda i: (0, 0))


def _bspec():
    return pl.BlockSpec((1, H), lambda i: (0, 0))


def _ab_body(h, wd, ws, a, b):
    hh = h[...]
    a[...] = _dot(hh, wd[...])
    b[...] = _dot(hh, ws[...])


def _ab(h, w1):
    return pl.pallas_call(
        _ab_body,
        grid=(NNODE // BN,),
        in_specs=[_hspec(), _wspec(), _wspec()],
        out_specs=[_hspec(), _hspec()],
        out_shape=[jax.ShapeDtypeStruct((NNODE, H), jnp.float32)] * 2,
    )(h, w1[:H], w1[H:2 * H])


def _deg_dis(da, db):
    deg = (da[...] + db[...])[:, :1]
    dis = jnp.where(deg > 0, lax.rsqrt(deg), 0.0)
    return deg, dis


def _post_body(sa, sb, da, db, w2, b2, tw0, g, hs):
    deg, dis = _deg_dis(da, db)
    h1 = jnp.maximum(_dot(sa[...] + sb[...], w2[...]) + deg * b2[...], 0.0)
    g[...] = _dot(h1, tw0[...])
    hs[...] = dis * h1


def _post(sa, sb, da, db, w2, b2, tw0):
    return pl.pallas_call(
        _post_body,
        grid=(NNODE // BN,),
        in_specs=[_hspec()] * 4 + [_wspec(), _bspec(), _wspec()],
        out_specs=[_hspec(), _hspec()],
        out_shape=[jax.ShapeDtypeStruct((NNODE, H), jnp.float32)] * 2,
    )(sa, sb, da, db, w2, b2.reshape(1, H), tw0)


def _mid_body(pa, pb, da, db, tw1, term, hs2):
    _, dis = _deg_dis(da, db)
    hp = dis * (pa[...] + pb[...])
    term[...] = _dot(hp, tw1[...])
    hs2[...] = dis * hp


def _mid(pa, pb, da, db, tw1):
    return pl.pallas_call(
        _mid_body,
        grid=(NNODE // BN,),
        in_specs=[_hspec()] * 4 + [_wspec()],
        out_specs=[_hspec(), _hspec()],
        out_shape=[jax.ShapeDtypeStruct((NNODE, H), jnp.float32)] * 2,
    )(pa, pb, da, db, tw1)


def _tagfin_body(pa, pb, da, db, g, term, tw2, tb, wd, ws, a, b):
    _, dis = _deg_dis(da, db)
    hp = dis * (pa[...] + pb[...])
    ht = jnp.maximum(g[...] + term[...] + _dot(hp, tw2[...]) + tb[...], 0.0)
    a[...] = _dot(ht, wd[...])
    b[...] = _dot(ht, ws[...])


def _tagfin(pa, pb, da, db, g, term, tw2, tb, w1_next):
    return pl.pallas_call(
        _tagfin_body,
        grid=(NNODE // BN,),
        in_specs=[_hspec()] * 6 + [_wspec(), _bspec(), _wspec(), _wspec()],
        out_specs=[_hspec(), _hspec()],
        out_shape=[jax.ShapeDtypeStruct((NNODE, H), jnp.float32)] * 2,
    )(pa, pb, da, db, g, term, tw2, tb.reshape(1, H), w1_next[:H],
      w1_next[H:2 * H])


def _final_body(sa, sb, da, db, w2, b2, out):
    deg, _ = _deg_dis(da, db)
    out[...] = _dot(sa[...] + sb[...], w2[...]) + deg * b2[...]


def _final(sa, sb, da, db, w2, b2):
    return pl.pallas_call(
        _final_body,
        grid=(NNODE // BN,),
        in_specs=[_hspec()] * 4 + [_wspec(), _bspec()],
        out_specs=_hspec(),
        out_shape=jax.ShapeDtypeStruct((NNODE, H), jnp.float32),
    )(sa, sb, da, db, w2, b2.reshape(1, H))


def kernel(x, edge_index, edge_attr,
           l0_w1, l0_b1, l0_w2, l0_b2,
           t1_w0, t1_w1, t1_w2, t1_b,
           l2_w1, l2_b1, l2_w2, l2_b2,
           t3_w0, t3_w1, t3_w2, t3_b,
           l4_w1, l4_b1, l4_w2, l4_b2):
    h0 = jnp.pad(x[:, 4:4 + NF], ((0, NNODE - N), (0, 0)))

    def pad_edges(v, fill):
        return jnp.pad(v.reshape(NS, EPW), ((0, 0), (0, RPW * KB - EPW)),
                       constant_values=fill).reshape(NS, RPW, KB)

    srcm = pad_edges(edge_index[0], N)
    dstm = pad_edges(edge_index[1], N)

    m2 = _directed_flag(edge_index)
    ones_n = jnp.ones((NNODE, H), jnp.float32)
    zn = jnp.zeros((NNODE, H), jnp.float32)

    cidx_base = pad_edges(jnp.arange(E, dtype=jnp.int32), 0)
    cidx_hot = jnp.broadcast_to(jnp.arange(KB, dtype=jnp.int32),
                                (NS, RPW, KB))

    def coef(v):
        return jnp.full((16,), v, jnp.float32)

    MSG = (coef(1.0), coef(1.0), coef(1.0), coef(0.0))
    HOP = (coef(0.0), coef(0.0), coef(0.0), coef(1.0))

    cbig = _prep_c(edge_attr, [(l0_w1[2 * H:], l0_b1), (l2_w1[2 * H:], l2_b1),
                               (l4_w1[2 * H:], l4_b1)])
    a0, b0 = _ab(h0, l0_w1)

    edge_op = _make_edge_op()

    def split(p):
        return p[:NNODE], p[NNODE:]

    def st0(p, g, term, da, db):
        return (ones_n, ones_n, cidx_hot) + HOP + (g, term, da, db)

    def st1(p, g, term, da, db):
        da, db = split(p)
        return (a0, b0, cidx_base) + MSG + (g, term, da, db)

    def _post_stage(p, g, term, da, db, w2, b2, tw0):
        g, hs = _post(p[:NNODE], p[NNODE:], da, db, w2, b2, tw0)
        return (hs, hs, cidx_hot) + HOP + (g, term, da, db)

    def _mid_stage(p, g, term, da, db, tw1):
        term, hs2 = _mid(p[:NNODE], p[NNODE:], da, db, tw1)
        return (hs2, hs2, cidx_hot) + HOP + (g, term, da, db)

    def _fin_stage(p, g, term, da, db, tw2, tb, w1n, lyr):
        a, b = _tagfin(p[:NNODE], p[NNODE:], da, db, g, term, tw2, tb, w1n)
        return (a, b, cidx_base + lyr * E) + MSG + (g, term, da, db)

    branches = [
        st0,
        st1,
        lambda p, g, t, da, db: _post_stage(p, g, t, da, db, l0_w2, l0_b2,
                                            t1_w0),
        lambda p, g, t, da, db: _mid_stage(p, g, t, da, db, t1_w1),
        lambda p, g, t, da, db: _fin_stage(p, g, t, da, db, t1_w2, t1_b,
                                           l2_w1, 1),
        lambda p, g, t, da, db: _post_stage(p, g, t, da, db, l2_w2, l2_b2,
                                            t3_w0),
        lambda p, g, t, da, db: _mid_stage(p, g, t, da, db, t3_w1),
        lambda p, g, t, da, db: _fin_stage(p, g, t, da, db, t3_w2, t3_b,
                                           l4_w1, 2),
    ]

    def body(carry, i):
        p, g, term, da, db = carry
        a, b, cidx, ga, gc, al, be, g, term, da, db = lax.switch(
            i, branches, p, g, term, da, db)
        p = edge_op(a, b, cbig, srcm, dstm, cidx, m2, ga, gc, al, be)
        return (p, g, term, da, db), None

    carry0 = (jnp.zeros((NC * NNODE, H), jnp.float32), zn, zn, zn, zn)
    (p, _, _, da, db), _ = lax.scan(body, carry0,
                                    jnp.arange(8, dtype=jnp.int32))
    return _final(p[:NNODE], p[NNODE:], da, db, l4_w2, l4_b2)[:N]
